# baseline (speedup 1.0000x reference)
import functools

import jax
import jax.numpy as jnp
from jax.experimental import pallas as pl
from jax.experimental.pallas import tpu as pltpu

_LANE = 128
_VMEM_LIMIT = 60000 * 1024


def _ceil_to(n, m):
    return ((n + m - 1) // m) * m


def _fused_kernel(a0_ref, a1_ref, a2_ref, a3_ref, x_ref, wpre_ref, bpre_ref,
                  wmean_ref, bmean_ref, wp_ref, bp_ref, o_ref,
                  a16_s, m_s, m16_s, z1h_s, *, tm, nq):
    s = pl.program_id(0)
    i = pl.program_id(1)
    f32 = jnp.float32

    @pl.when((s == 0) & (i == 0))
    def _mlp():
        h = jnp.maximum(
            jnp.dot(x_ref[...], wpre_ref[...], preferred_element_type=f32)
            + bpre_ref[...], 0.0)
        m = jnp.dot(h, wmean_ref[...], preferred_element_type=f32) + bmean_ref[...]
        m_s[...] = m
        m16_s[...] = m.astype(jnp.bfloat16)

    @pl.when(s == 0)
    def _step1():
        acc = 0.5 * m_s[pl.ds(i * tm, tm), :]
        for q, a_ref in enumerate((a0_ref, a1_ref, a2_ref, a3_ref)):
            q16 = (0.5 * a_ref[...]).astype(jnp.bfloat16)
            a16_s[pl.ds(i * tm, tm), pl.ds(q * nq, nq)] = q16
            acc = acc + jnp.dot(q16, m16_s[pl.ds(q * nq, nq), :],
                                preferred_element_type=f32)
        z1h_s[pl.ds(i * tm, tm), :] = acc.astype(jnp.bfloat16)

    @pl.when(s == 1)
    def _step2():
        r = pl.num_programs(1) - 1 - i
        a16 = a16_s[pl.ds(r * tm, tm), :]
        z2 = (jnp.dot(a16, z1h_s[...], preferred_element_type=f32)
              + 0.5 * m_s[pl.ds(r * tm, tm), :])
        o_ref[...] = (jnp.dot(z2, wp_ref[...], preferred_element_type=f32)
                      + bp_ref[...])


def kernel(x, a_hat, w_pre, b_pre, w_mean, b_mean, w_proj, b_proj,
           bn_gamma, bn_beta, bn_rmean, bn_rvar):
    n, fin = x.shape
    hid = w_pre.shape[1]
    out_ch = w_proj.shape[1]
    f32 = jnp.float32

    fp = _ceil_to(fin, _LANE)
    hp = _ceil_to(hid, _LANE)
    op = _ceil_to(out_ch, _LANE)

    tm = 512
    n_pad = _ceil_to(max(n, 2 * tm), 2 * tm)
    grid_r = n_pad // tm
    nq = n_pad // 4

    def pad2(arr, r, c):
        if arr.shape == (r, c):
            return arr
        return jnp.pad(arr, ((0, r - arr.shape[0]), (0, c - arr.shape[1])))

    a_p = pad2(a_hat, n_pad, n_pad)
    x_p = pad2(x.astype(f32), n_pad, fp)
    w_pre_p = pad2(w_pre, fp, hp)
    b_pre_p = pad2(b_pre, 1, hp)
    w_mean_p = pad2(w_mean, hp, op)
    b_mean_p = pad2(b_mean, 1, op)

    eps = 1e-5
    scale = bn_gamma[0] * jax.lax.rsqrt(bn_rvar[0] + eps)
    shift = bn_beta[0] - bn_rmean[0] * scale
    w_proj_f = pad2(scale[:, None] * w_proj, op, op)
    b_proj_f = pad2(b_proj + (shift @ w_proj)[None, :], 1, op)

    out_p = pl.pallas_call(
        functools.partial(_fused_kernel, tm=tm, nq=nq),
        out_shape=jax.ShapeDtypeStruct((n_pad, op), f32),
        grid=(2, grid_r),
        in_specs=[
            pl.BlockSpec((tm, nq),
                         lambda s, i: (i * (1 - s) + (grid_r - 1) * s, 0)),
            pl.BlockSpec((tm, nq),
                         lambda s, i: (i * (1 - s) + (grid_r - 1) * s, 1)),
            pl.BlockSpec((tm, nq),
                         lambda s, i: (i * (1 - s) + (grid_r - 1) * s, 2)),
            pl.BlockSpec((tm, nq),
                         lambda s, i: (i * (1 - s) + (grid_r - 1) * s, 3)),
            pl.BlockSpec((n_pad, fp), lambda s, i: (0, 0)),
            pl.BlockSpec((fp, hp), lambda s, i: (0, 0)),
            pl.BlockSpec((1, hp), lambda s, i: (0, 0)),
            pl.BlockSpec((hp, op), lambda s, i: (0, 0)),
            pl.BlockSpec((1, op), lambda s, i: (0, 0)),
            pl.BlockSpec((op, op), lambda s, i: (0, 0)),
            pl.BlockSpec((1, op), lambda s, i: (0, 0)),
        ],
        out_specs=pl.BlockSpec(
            (tm, op),
            lambda s, i: ((grid_r - 1) * (1 - s) + (grid_r - 1 - i) * s, 0)),
        scratch_shapes=[
            pltpu.VMEM((n_pad, n_pad), jnp.bfloat16),
            pltpu.VMEM((n_pad, op), f32),
            pltpu.VMEM((n_pad, op), jnp.bfloat16),
            pltpu.VMEM((n_pad, op), jnp.bfloat16),
        ],
        compiler_params=pltpu.CompilerParams(
            dimension_semantics=("arbitrary", "arbitrary"),
            vmem_limit_bytes=_VMEM_LIMIT),
    )(a_p, a_p, a_p, a_p, x_p, w_pre_p, b_pre_p, w_mean_p, b_mean_p,
      w_proj_f, b_proj_f)

    return out_p[:n, :out_ch]


# speedup vs baseline: 1.0236x; 1.0236x over previous
import functools

import jax
import jax.numpy as jnp
from jax.experimental import pallas as pl
from jax.experimental.pallas import tpu as pltpu

_LANE = 128
_VMEM_LIMIT = 60000 * 1024


def _ceil_to(n, m):
    return ((n + m - 1) // m) * m


def _fused_kernel(al_ref, ar_ref, x_ref, wpre_ref, bpre_ref, wmean_ref,
                  bmean_ref, wp_ref, bp_ref, o_ref,
                  a16_s, m_s, m16_s, z1h_s, *, tm, nh):
    s = pl.program_id(0)
    i = pl.program_id(1)
    f32 = jnp.float32

    @pl.when((s == 0) & (i == 0))
    def _mlp():
        h = jnp.maximum(
            jnp.dot(x_ref[...], wpre_ref[...], preferred_element_type=f32)
            + bpre_ref[...], 0.0)
        m = jnp.dot(h, wmean_ref[...], preferred_element_type=f32) + bmean_ref[...]
        m_s[...] = m
        m16_s[...] = m.astype(jnp.bfloat16)

    @pl.when(s == 0)
    def _step1():
        l16 = (0.5 * al_ref[...]).astype(jnp.bfloat16)
        r16 = (0.5 * ar_ref[...]).astype(jnp.bfloat16)
        a16_s[pl.ds(i * tm, tm), :nh] = l16
        a16_s[pl.ds(i * tm, tm), nh:] = r16
        z1 = (jnp.dot(l16, m16_s[:nh, :], preferred_element_type=f32)
              + jnp.dot(r16, m16_s[nh:, :], preferred_element_type=f32)
              + 0.5 * m_s[pl.ds(i * tm, tm), :])
        z1h_s[pl.ds(i * tm, tm), :] = z1.astype(jnp.bfloat16)

    @pl.when(s == 1)
    def _step2():
        r = pl.num_programs(1) - 1 - i
        a16 = a16_s[pl.ds(r * tm, tm), :]
        z2 = (jnp.dot(a16, z1h_s[...], preferred_element_type=f32)
              + 0.5 * m_s[pl.ds(r * tm, tm), :])
        o_ref[...] = (jnp.dot(z2, wp_ref[...], preferred_element_type=f32)
                      + bp_ref[...])


def kernel(x, a_hat, w_pre, b_pre, w_mean, b_mean, w_proj, b_proj,
           bn_gamma, bn_beta, bn_rmean, bn_rvar):
    n, fin = x.shape
    hid = w_pre.shape[1]
    out_ch = w_proj.shape[1]
    f32 = jnp.float32

    fp = _ceil_to(fin, _LANE)
    hp = _ceil_to(hid, _LANE)
    op = _ceil_to(out_ch, _LANE)

    tm = 512
    n_pad = _ceil_to(max(n, 2 * tm), 2 * tm)
    grid_r = n_pad // tm
    nh = n_pad // 2

    def pad2(arr, r, c):
        if arr.shape == (r, c):
            return arr
        return jnp.pad(arr, ((0, r - arr.shape[0]), (0, c - arr.shape[1])))

    a_p = pad2(a_hat, n_pad, n_pad)
    x_p = pad2(x.astype(f32), n_pad, fp)
    w_pre_p = pad2(w_pre, fp, hp)
    b_pre_p = pad2(b_pre, 1, hp)
    w_mean_p = pad2(w_mean, hp, op)
    b_mean_p = pad2(b_mean, 1, op)

    eps = 1e-5
    scale = bn_gamma[0] * jax.lax.rsqrt(bn_rvar[0] + eps)
    shift = bn_beta[0] - bn_rmean[0] * scale
    w_proj_f = pad2(scale[:, None] * w_proj, op, op)
    b_proj_f = pad2(b_proj + (shift @ w_proj)[None, :], 1, op)

    out_p = pl.pallas_call(
        functools.partial(_fused_kernel, tm=tm, nh=nh),
        out_shape=jax.ShapeDtypeStruct((n_pad, op), f32),
        grid=(2, grid_r),
        in_specs=[
            pl.BlockSpec((tm, nh),
                         lambda s, i: (i * (1 - s) + (grid_r - 1) * s, 0)),
            pl.BlockSpec((tm, nh),
                         lambda s, i: (i * (1 - s) + (grid_r - 1) * s, 1)),
            pl.BlockSpec((n_pad, fp), lambda s, i: (0, 0)),
            pl.BlockSpec((fp, hp), lambda s, i: (0, 0)),
            pl.BlockSpec((1, hp), lambda s, i: (0, 0)),
            pl.BlockSpec((hp, op), lambda s, i: (0, 0)),
            pl.BlockSpec((1, op), lambda s, i: (0, 0)),
            pl.BlockSpec((op, op), lambda s, i: (0, 0)),
            pl.BlockSpec((1, op), lambda s, i: (0, 0)),
        ],
        out_specs=pl.BlockSpec(
            (tm, op),
            lambda s, i: ((grid_r - 1) * (1 - s) + (grid_r - 1 - i) * s, 0)),
        scratch_shapes=[
            pltpu.VMEM((n_pad, n_pad), jnp.bfloat16),
            pltpu.VMEM((n_pad, op), f32),
            pltpu.VMEM((n_pad, op), jnp.bfloat16),
            pltpu.VMEM((n_pad, op), jnp.bfloat16),
        ],
        compiler_params=pltpu.CompilerParams(
            dimension_semantics=("arbitrary", "arbitrary"),
            vmem_limit_bytes=_VMEM_LIMIT),
    )(a_p, a_p, x_p, w_pre_p, b_pre_p, w_mean_p, b_mean_p, w_proj_f, b_proj_f)

    return out_p[:n, :out_ch]


# speedup vs baseline: 1.0443x; 1.0202x over previous
import functools

import jax
import jax.numpy as jnp
from jax.experimental import pallas as pl
from jax.experimental.pallas import tpu as pltpu

_LANE = 128
_VMEM_LIMIT = 60000 * 1024


def _ceil_to(n, m):
    return ((n + m - 1) // m) * m


def _fused_kernel(al_ref, ar_ref, x_ref, wpre_ref, bpre_ref, wmean_ref,
                  bmean_ref, wp_ref, bp_ref, o_ref,
                  a16_s, m16_s, z1h_s, z2t_s, *, tm, nh):
    s = pl.program_id(0)
    i = pl.program_id(1)
    f32 = jnp.float32

    @pl.when((s == 0) & (i == 0))
    def _mlp():
        h = jnp.maximum(
            jnp.dot(x_ref[...], wpre_ref[...], preferred_element_type=f32)
            + bpre_ref[...], 0.0)
        m = jnp.dot(h, wmean_ref[...], preferred_element_type=f32) + bmean_ref[...]
        m16_s[...] = m.astype(jnp.bfloat16)

    @pl.when(s == 0)
    def _step1():
        l16 = (0.5 * al_ref[...]).astype(jnp.bfloat16)
        r16 = (0.5 * ar_ref[...]).astype(jnp.bfloat16)
        a16_s[pl.ds(i * tm, tm), :nh] = l16
        a16_s[pl.ds(i * tm, tm), nh:] = r16
        z1 = (jnp.dot(l16, m16_s[:nh, :], preferred_element_type=f32)
              + jnp.dot(r16, m16_s[nh:, :], preferred_element_type=f32)
              + 0.5 * m16_s[pl.ds(i * tm, tm), :].astype(f32))
        z1h_s[pl.ds(i * tm, tm), :] = z1.astype(jnp.bfloat16)

        gr2 = pl.num_programs(1) // 2

        @pl.when(i >= gr2)
        def _step2_top_half():
            z2t_s[pl.ds((i - gr2) * tm, tm), :] = jnp.dot(
                l16, z1h_s[:nh, :], preferred_element_type=f32)

    @pl.when(s == 1)
    def _step2():
        gr = pl.num_programs(1)
        gr2 = gr // 2
        r = gr - 1 - i
        res = 0.5 * m16_s[pl.ds(r * tm, tm), :].astype(f32)

        @pl.when(r >= gr2)
        def _corr_bottom_only():
            z2 = (z2t_s[pl.ds((r - gr2) * tm, tm), :]
                  + jnp.dot(a16_s[pl.ds(r * tm, tm), nh:], z1h_s[nh:, :],
                            preferred_element_type=f32)
                  + res)
            o_ref[...] = (jnp.dot(z2, wp_ref[...], preferred_element_type=f32)
                          + bp_ref[...])

        @pl.when(r < gr2)
        def _full():
            z2 = (jnp.dot(a16_s[pl.ds(r * tm, tm), :], z1h_s[...],
                          preferred_element_type=f32)
                  + res)
            o_ref[...] = (jnp.dot(z2, wp_ref[...], preferred_element_type=f32)
                          + bp_ref[...])


def kernel(x, a_hat, w_pre, b_pre, w_mean, b_mean, w_proj, b_proj,
           bn_gamma, bn_beta, bn_rmean, bn_rvar):
    n, fin = x.shape
    hid = w_pre.shape[1]
    out_ch = w_proj.shape[1]
    f32 = jnp.float32

    fp = _ceil_to(fin, _LANE)
    hp = _ceil_to(hid, _LANE)
    op = _ceil_to(out_ch, _LANE)

    tm = 512
    n_pad = _ceil_to(max(n, 2 * tm), 2 * tm)
    grid_r = n_pad // tm
    nh = n_pad // 2

    def pad2(arr, r, c):
        if arr.shape == (r, c):
            return arr
        return jnp.pad(arr, ((0, r - arr.shape[0]), (0, c - arr.shape[1])))

    a_p = pad2(a_hat, n_pad, n_pad)
    x_p = pad2(x.astype(f32), n_pad, fp)
    w_pre_p = pad2(w_pre, fp, hp)
    b_pre_p = pad2(b_pre, 1, hp)
    w_mean_p = pad2(w_mean, hp, op)
    b_mean_p = pad2(b_mean, 1, op)

    eps = 1e-5
    scale = bn_gamma[0] * jax.lax.rsqrt(bn_rvar[0] + eps)
    shift = bn_beta[0] - bn_rmean[0] * scale
    w_proj_f = pad2(scale[:, None] * w_proj, op, op)
    b_proj_f = pad2(b_proj + (shift @ w_proj)[None, :], 1, op)

    out_p = pl.pallas_call(
        functools.partial(_fused_kernel, tm=tm, nh=nh),
        out_shape=jax.ShapeDtypeStruct((n_pad, op), f32),
        grid=(2, grid_r),
        in_specs=[
            pl.BlockSpec((tm, nh),
                         lambda s, i: (i * (1 - s) + (grid_r - 1) * s, 0)),
            pl.BlockSpec((tm, nh),
                         lambda s, i: (i * (1 - s) + (grid_r - 1) * s, 1)),
            pl.BlockSpec((n_pad, fp), lambda s, i: (0, 0)),
            pl.BlockSpec((fp, hp), lambda s, i: (0, 0)),
            pl.BlockSpec((1, hp), lambda s, i: (0, 0)),
            pl.BlockSpec((hp, op), lambda s, i: (0, 0)),
            pl.BlockSpec((1, op), lambda s, i: (0, 0)),
            pl.BlockSpec((op, op), lambda s, i: (0, 0)),
            pl.BlockSpec((1, op), lambda s, i: (0, 0)),
        ],
        out_specs=pl.BlockSpec(
            (tm, op),
            lambda s, i: ((grid_r - 1) * (1 - s) + (grid_r - 1 - i) * s, 0)),
        scratch_shapes=[
            pltpu.VMEM((n_pad, n_pad), jnp.bfloat16),
            pltpu.VMEM((n_pad, op), jnp.bfloat16),
            pltpu.VMEM((n_pad, op), jnp.bfloat16),
            pltpu.VMEM((nh, op), f32),
        ],
        compiler_params=pltpu.CompilerParams(
            dimension_semantics=("arbitrary", "arbitrary"),
            vmem_limit_bytes=_VMEM_LIMIT),
    )(a_p, a_p, x_p, w_pre_p, b_pre_p, w_mean_p, b_mean_p, w_proj_f, b_proj_f)

    return out_p[:n, :out_ch]
